# baseline (speedup 1.0000x reference)
import functools

import jax
import jax.numpy as jnp
from jax import lax
from jax.experimental import pallas as pl
from jax.experimental.pallas import tpu as pltpu
from jax.experimental.pallas import tpu_sc as plsc

_N = 10000
_E = 320000
_DE = 16
_DN = 128
_NC = 2
_NS = 16
_NP = 10240
_ER = _E // 128
_CRC = 1248
_CR = 208
_NCH = _CRC // _CR
_TR = _ER - _NC * _CRC


def _sc_segment_partials(col2, eat3):
    mesh = plsc.VectorSubcoreMesh(core_axis_name="c", subcore_axis_name="s")

    @functools.partial(
        pl.kernel,
        out_type=jax.ShapeDtypeStruct((_NC, _NS, _NP), jnp.float32),
        mesh=mesh,
        scratch_types=[
            pltpu.VMEM((2, _CR, 128), jnp.int32),
            pltpu.VMEM((2, _CR, 128), jnp.float32),
            pltpu.VMEM((_NP,), jnp.float32),
            pltpu.SemaphoreType.DMA,
            pltpu.SemaphoreType.DMA,
        ],
        compiler_params=pltpu.CompilerParams(
            needs_layout_passes=False, use_tc_tiling_on_sc=False),
    )
    def run(col_hbm, val_hbm, out_hbm, idx_v, val_v, acc, ld_sem0, ld_sem1):
        cid = lax.axis_index("c")
        sid = lax.axis_index("s")

        def fire(ch, buf):
            row0 = cid * _CRC + ch * _CR
            sem = ld_sem0 if buf == 0 else ld_sem1
            ld_i = pltpu.async_copy(col_hbm.at[pl.ds(row0, _CR), 1, :],
                                    idx_v.at[buf], sem)
            ld_v = pltpu.async_copy(
                val_hbm.at[sid // 8, pl.ds(row0, _CR), sid % 8, :],
                val_v.at[buf], sem)
            return ld_i, ld_v

        def make_row_body(buf):
            def row_body(r):
                for c in range(8):
                    iv = idx_v[buf, r, pl.ds(c * 16, 16)]
                    vv = val_v[buf, r, pl.ds(c * 16, 16)]
                    plsc.addupdate_scatter(acc, [iv], vv)
            return row_body

        pending = fire(0, 0)

        @plsc.parallel_loop(0, _NP // 16, 1, unroll=8)
        def zero_body(i):
            acc[pl.ds(i * 16, 16)] = jnp.zeros((16,), jnp.float32)

        for ch in range(_NCH):
            buf = ch % 2
            ld_i, ld_v = pending
            if ch + 1 < _NCH:
                nxt = fire(ch + 1, (ch + 1) % 2)
            ld_i.wait()
            ld_v.wait()
            if ch + 1 < _NCH:
                pending = nxt
            plsc.parallel_loop(0, _CR, 1, unroll=8)(make_row_body(buf))

        @pl.when(cid == _NC - 1)
        def _tail():
            ld_i = pltpu.async_copy(col_hbm.at[pl.ds(_NC * _CRC, _TR), 1, :],
                                    idx_v.at[0, pl.ds(0, _TR), :], ld_sem0)
            ld_v = pltpu.async_copy(
                val_hbm.at[sid // 8, pl.ds(_NC * _CRC, _TR), sid % 8, :],
                val_v.at[0, pl.ds(0, _TR), :], ld_sem0)
            ld_i.wait()
            ld_v.wait()
            plsc.parallel_loop(0, _TR, 1)(make_row_body(0))

        pltpu.sync_copy(acc, out_hbm.at[cid, sid])

    return run(col2, eat3)


_BN = 1024


def _mlp_body(x_ref, p_ref, w1x_ref, w1a_ref, b1_ref, w2_ref, b2_ref,
              w3_ref, b3_ref, g_ref, be_ref, o_ref):
    xb = x_ref[...]
    aggt = p_ref[0] + p_ref[1]
    h = jnp.dot(xb, w1x_ref[...], preferred_element_type=jnp.float32)
    h = h + lax.dot_general(aggt, w1a_ref[...], (((0,), (0,)), ((), ())),
                            preferred_element_type=jnp.float32)
    h = jnp.maximum(h + b1_ref[...], 0.0)
    h = jnp.maximum(
        jnp.dot(h, w2_ref[...], preferred_element_type=jnp.float32) + b2_ref[...],
        0.0)
    h = jnp.dot(h, w3_ref[...], preferred_element_type=jnp.float32) + b3_ref[...]
    mu = jnp.mean(h, axis=-1, keepdims=True)
    d = h - mu
    var = jnp.mean(d * d, axis=-1, keepdims=True)
    hn = d * lax.rsqrt(var + 1e-5) * g_ref[...] + be_ref[...]
    o_ref[...] = hn + xb


def _tc_mlp(x, parts, w1x, w1a, b1, W2, b2, W3, b3, g, be):
    full = lambda i: (0, 0)
    return pl.pallas_call(
        _mlp_body,
        grid=(pl.cdiv(_N, _BN),),
        in_specs=[
            pl.BlockSpec((_BN, _DN), lambda i: (i, 0)),
            pl.BlockSpec((_NC, _NS, _BN), lambda i: (0, 0, i)),
            pl.BlockSpec((_DN, _DN), full),
            pl.BlockSpec((_DE, _DN), full),
            pl.BlockSpec((1, _DN), full),
            pl.BlockSpec((_DN, _DN), full),
            pl.BlockSpec((1, _DN), full),
            pl.BlockSpec((_DN, _DN), full),
            pl.BlockSpec((1, _DN), full),
            pl.BlockSpec((1, _DN), full),
            pl.BlockSpec((1, _DN), full),
        ],
        out_specs=pl.BlockSpec((_BN, _DN), lambda i: (i, 0)),
        out_shape=jax.ShapeDtypeStruct((_N, _DN), jnp.float32),
    )(x, parts, w1x, w1a, b1, W2, b2, W3, b3, g, be)


def kernel(x, edge_index, edge_attr, W1, b1, W2, b2, W3, b3, ln_gamma, ln_beta):
    ei3 = edge_index.reshape(2, _E // 128, 128).transpose(1, 0, 2)
    eat4 = edge_attr.reshape(_E // 128, 128, 2, 8).transpose(2, 0, 3, 1)
    parts = _sc_segment_partials(ei3, eat4)
    return _tc_mlp(
        x, parts, W1[:_DN], W1[_DN:], b1.reshape(1, -1), W2, b2.reshape(1, -1),
        W3, b3.reshape(1, -1), ln_gamma.reshape(1, -1), ln_beta.reshape(1, -1))


# speedup vs baseline: 1.0364x; 1.0364x over previous
import functools

import jax
import jax.numpy as jnp
from jax import lax
from jax.experimental import pallas as pl
from jax.experimental.pallas import tpu as pltpu
from jax.experimental.pallas import tpu_sc as plsc

_N = 10000
_E = 320000
_DE = 16
_DN = 128
_NC = 2
_NS = 16
_NP = 10240
_ER = _E // 128
_CRC = 1248
_CR = 208
_NCH = _CRC // _CR
_TR = _ER - _NC * _CRC


def _sc_segment_partials(col2, eat3):
    mesh = plsc.VectorSubcoreMesh(core_axis_name="c", subcore_axis_name="s")

    @functools.partial(
        pl.kernel,
        out_type=jax.ShapeDtypeStruct((_NC, _NS, _NP), jnp.float32),
        mesh=mesh,
        scratch_types=[
            pltpu.VMEM((2, _CR, 128), jnp.int32),
            pltpu.VMEM((2, _CR, 128), jnp.float32),
            pltpu.VMEM((_NP,), jnp.float32),
            pltpu.SemaphoreType.DMA,
            pltpu.SemaphoreType.DMA,
        ],
        compiler_params=pltpu.CompilerParams(
            needs_layout_passes=False, use_tc_tiling_on_sc=False),
    )
    def run(col_hbm, val_hbm, out_hbm, idx_v, val_v, acc, ld_sem0, ld_sem1):
        cid = lax.axis_index("c")
        sid = lax.axis_index("s")

        def fire(ch, buf):
            row0 = cid * _CRC + ch * _CR
            sem = ld_sem0 if buf == 0 else ld_sem1
            ld_i = pltpu.async_copy(col_hbm.at[pl.ds(row0, _CR), 1, :],
                                    idx_v.at[buf], sem)
            ld_v = pltpu.async_copy(
                val_hbm.at[sid // 8, pl.ds(row0, _CR), sid % 8, :],
                val_v.at[buf], sem)
            return ld_i, ld_v

        def make_row_body(buf):
            def row_body(r):
                for c in range(8):
                    iv = idx_v[buf, r, pl.ds(c * 16, 16)]
                    vv = val_v[buf, r, pl.ds(c * 16, 16)]
                    plsc.addupdate_scatter(acc, [iv], vv)
            return row_body

        pending = fire(0, 0)

        @plsc.parallel_loop(0, _NP // 16, 1, unroll=8)
        def zero_body(i):
            acc[pl.ds(i * 16, 16)] = jnp.zeros((16,), jnp.float32)

        for ch in range(_NCH):
            buf = ch % 2
            ld_i, ld_v = pending
            if ch + 1 < _NCH:
                nxt = fire(ch + 1, (ch + 1) % 2)
            ld_i.wait()
            ld_v.wait()
            if ch + 1 < _NCH:
                pending = nxt
            plsc.parallel_loop(0, _CR, 1, unroll=4)(make_row_body(buf))

        @pl.when(cid == _NC - 1)
        def _tail():
            ld_i = pltpu.async_copy(col_hbm.at[pl.ds(_NC * _CRC, _TR), 1, :],
                                    idx_v.at[0, pl.ds(0, _TR), :], ld_sem0)
            ld_v = pltpu.async_copy(
                val_hbm.at[sid // 8, pl.ds(_NC * _CRC, _TR), sid % 8, :],
                val_v.at[0, pl.ds(0, _TR), :], ld_sem0)
            ld_i.wait()
            ld_v.wait()
            plsc.parallel_loop(0, _TR, 1)(make_row_body(0))

        pltpu.sync_copy(acc, out_hbm.at[cid, sid])

    return run(col2, eat3)


_BN = 1024


def _mlp_body(x_ref, p_ref, w1x_ref, w1a_ref, b1_ref, w2_ref, b2_ref,
              w3_ref, b3_ref, g_ref, be_ref, o_ref):
    xb = x_ref[...]
    aggt = p_ref[0] + p_ref[1]
    h = jnp.dot(xb, w1x_ref[...], preferred_element_type=jnp.float32)
    h = h + lax.dot_general(aggt, w1a_ref[...], (((0,), (0,)), ((), ())),
                            preferred_element_type=jnp.float32)
    h = jnp.maximum(h + b1_ref[...], 0.0)
    h = jnp.maximum(
        jnp.dot(h, w2_ref[...], preferred_element_type=jnp.float32) + b2_ref[...],
        0.0)
    h = jnp.dot(h, w3_ref[...], preferred_element_type=jnp.float32) + b3_ref[...]
    mu = jnp.mean(h, axis=-1, keepdims=True)
    d = h - mu
    var = jnp.mean(d * d, axis=-1, keepdims=True)
    hn = d * lax.rsqrt(var + 1e-5) * g_ref[...] + be_ref[...]
    o_ref[...] = hn + xb


def _tc_mlp(x, parts, w1x, w1a, b1, W2, b2, W3, b3, g, be):
    full = lambda i: (0, 0)
    return pl.pallas_call(
        _mlp_body,
        grid=(pl.cdiv(_N, _BN),),
        in_specs=[
            pl.BlockSpec((_BN, _DN), lambda i: (i, 0)),
            pl.BlockSpec((_NC, _NS, _BN), lambda i: (0, 0, i)),
            pl.BlockSpec((_DN, _DN), full),
            pl.BlockSpec((_DE, _DN), full),
            pl.BlockSpec((1, _DN), full),
            pl.BlockSpec((_DN, _DN), full),
            pl.BlockSpec((1, _DN), full),
            pl.BlockSpec((_DN, _DN), full),
            pl.BlockSpec((1, _DN), full),
            pl.BlockSpec((1, _DN), full),
            pl.BlockSpec((1, _DN), full),
        ],
        out_specs=pl.BlockSpec((_BN, _DN), lambda i: (i, 0)),
        out_shape=jax.ShapeDtypeStruct((_N, _DN), jnp.float32),
    )(x, parts, w1x, w1a, b1, W2, b2, W3, b3, g, be)


def kernel(x, edge_index, edge_attr, W1, b1, W2, b2, W3, b3, ln_gamma, ln_beta):
    ei3 = edge_index.reshape(2, _E // 128, 128).transpose(1, 0, 2)
    eat4 = edge_attr.reshape(_E // 128, 128, 2, 8).transpose(2, 0, 3, 1)
    parts = _sc_segment_partials(ei3, eat4)
    return _tc_mlp(
        x, parts, W1[:_DN], W1[_DN:], b1.reshape(1, -1), W2, b2.reshape(1, -1),
        W3, b3.reshape(1, -1), ln_gamma.reshape(1, -1), ln_beta.reshape(1, -1))
